# Initial kernel scaffold; baseline (speedup 1.0000x reference)
import jax
import jax.numpy as jnp
from jax.experimental import pallas as pl


def kernel(x, edge_index, edge_vals, W, b):
    raise NotImplementedError("write your pallas kernel here")


# speedup vs baseline: 3.0766x; 3.0766x over previous
import functools

import jax
import jax.numpy as jnp
from jax import lax
from jax.experimental import pallas as pl
from jax.experimental.pallas import tpu as pltpu
from jax.experimental.pallas import tpu_sc as plsc

N = 10000
E = 320000
D = 128
DH = D // 2

NC = 2
NS = 16
EPT = E // NS
C = 80
J = EPT // C
RPT = 632
N_PAD = RPT * NS


def _mm_body(x_ref, w_ref, o_ref):
    h = jnp.dot(x_ref[...], w_ref[...], preferred_element_type=jnp.float32)
    o_ref[0] = h[:, :DH]
    o_ref[1] = h[:, DH:]


def _matmul_split(x, W):
    blk = 1000
    grid = N // blk
    return pl.pallas_call(
        _mm_body,
        grid=(grid,),
        in_specs=[
            pl.BlockSpec((blk, D), lambda i: (i, 0)),
            pl.BlockSpec((D, D), lambda i: (0, 0)),
        ],
        out_specs=pl.BlockSpec((2, blk, DH), lambda i: (0, i, 0)),
        out_shape=jax.ShapeDtypeStruct((2, N, DH), jnp.float32),
    )(x, W)


def _combine_body(p0_ref, p1_ref, b_ref, o_ref):
    h = jnp.concatenate([p0_ref[0], p1_ref[0]], axis=1)
    o_ref[...] = jnp.maximum(h + b_ref[...], 0.0)


def _combine(partials, b2d):
    blk = 1000
    grid = N // blk
    return pl.pallas_call(
        _combine_body,
        grid=(grid,),
        in_specs=[
            pl.BlockSpec((1, blk, DH), lambda i: (0, i, 0)),
            pl.BlockSpec((1, blk, DH), lambda i: (1, i, 0)),
            pl.BlockSpec((1, D), lambda i: (0, 0)),
        ],
        out_specs=pl.BlockSpec((blk, D), lambda i: (i, 0)),
        out_shape=jax.ShapeDtypeStruct((N, D), jnp.float32),
    )(partials, partials, b2d)


def _lane_splat(vec, e):
    idx = jnp.full((16, 1), e, jnp.int32)
    return lax.gather(
        vec, idx,
        dimension_numbers=lax.GatherDimensionNumbers(
            offset_dims=(), collapsed_slice_dims=(0,), start_index_map=(0,)),
        slice_sizes=(1,),
        mode=lax.GatherScatterMode.PROMISE_IN_BOUNDS)


def _sc_body(h_hbm, src_hbm, dst_hbm, vals_hbm, zero_hbm, out_hbm,
             src_v, dst_v, vals_v, rows_v, acc_sh, sem):
    c = lax.axis_index("c")
    s = lax.axis_index("s")

    pltpu.sync_copy(zero_hbm.at[pl.ds(s * RPT, RPT)],
                    acc_sh.at[pl.ds(s * RPT, RPT)])
    pltpu.sync_copy(src_hbm.at[s], src_v)
    pltpu.sync_copy(dst_hbm.at[s], dst_v)
    pltpu.sync_copy(vals_hbm.at[s], vals_v)
    plsc.subcore_barrier()

    @pl.loop(0, J)
    def _chunk(j):
        pltpu.async_copy(h_hbm.at[c].at[src_v.at[j]], rows_v, sem).wait()

        @pl.loop(0, C // 16)
        def _grp(g):
            vgrp = vals_v[j, pl.ds(g * 16, 16)]
            for e in range(16):
                vsplat = _lane_splat(vgrp, e)
                row = g * 16 + e
                for d in range(DH // 16):
                    sl = pl.ds(d * 16, 16)
                    rows_v[row, sl] = rows_v[row, sl] * vsplat

        pltpu.sync_copy(rows_v, acc_sh.at[dst_v.at[j]], add=True)

    plsc.subcore_barrier()
    pltpu.sync_copy(acc_sh.at[pl.ds(s * RPT, RPT)],
                    out_hbm.at[c, pl.ds(s * RPT, RPT)])


_sc_kernel = functools.partial(
    pl.kernel,
    out_type=jax.ShapeDtypeStruct((NC, N_PAD, DH), jnp.float32),
    mesh=plsc.VectorSubcoreMesh(core_axis_name="c", subcore_axis_name="s",
                                num_cores=NC, num_subcores=NS),
    compiler_params=pltpu.CompilerParams(use_tc_tiling_on_sc=False),
    scratch_types=[
        pltpu.VMEM((J, C), jnp.int32),
        pltpu.VMEM((J, C), jnp.int32),
        pltpu.VMEM((J, C), jnp.float32),
        pltpu.VMEM((C, DH), jnp.float32),
        pltpu.VMEM_SHARED((N_PAD, DH), jnp.float32),
        pltpu.SemaphoreType.DMA,
    ],
)(_sc_body)


@jax.jit
def kernel(x, edge_index, edge_vals, W, b):
    h2 = _matmul_split(x, W)
    src = edge_index[0].reshape(NS, J, C)
    dst = edge_index[1].reshape(NS, J, C)
    vals = edge_vals.reshape(NS, J, C)
    zeros = jnp.zeros((N_PAD, DH), jnp.float32)
    partials = _sc_kernel(h2, src, dst, vals, zeros)
    return _combine(partials, b.reshape(1, D))


# speedup vs baseline: 5.0199x; 1.6316x over previous
import functools

import jax
import jax.numpy as jnp
from jax import lax
from jax.experimental import pallas as pl
from jax.experimental.pallas import tpu as pltpu
from jax.experimental.pallas import tpu_sc as plsc

N = 10000
E = 320000
D = 128
DH = D // 2

NC = 2
NS = 16
EPT = E // NS
C = 80
J = EPT // C
RPT = 632
N_PAD = RPT * NS


def _mm_body(x_ref, w_ref, o_ref):
    h = jnp.dot(x_ref[...], w_ref[...], preferred_element_type=jnp.float32)
    o_ref[0] = h[:, :DH]
    o_ref[1] = h[:, DH:]


def _matmul_split(x, W):
    blk = 1000
    grid = N // blk
    return pl.pallas_call(
        _mm_body,
        grid=(grid,),
        in_specs=[
            pl.BlockSpec((blk, D), lambda i: (i, 0)),
            pl.BlockSpec((D, D), lambda i: (0, 0)),
        ],
        out_specs=pl.BlockSpec((2, blk, DH), lambda i: (0, i, 0)),
        out_shape=jax.ShapeDtypeStruct((2, N, DH), jnp.float32),
    )(x, W)


def _combine_body(p0_ref, p1_ref, b_ref, o_ref):
    h = jnp.concatenate([p0_ref[0], p1_ref[0]], axis=1)
    o_ref[...] = jnp.maximum(h + b_ref[...], 0.0)


def _combine(partials, b2d):
    blk = 1000
    grid = N // blk
    return pl.pallas_call(
        _combine_body,
        grid=(grid,),
        in_specs=[
            pl.BlockSpec((1, blk, DH), lambda i: (0, i, 0)),
            pl.BlockSpec((1, blk, DH), lambda i: (1, i, 0)),
            pl.BlockSpec((1, D), lambda i: (0, 0)),
        ],
        out_specs=pl.BlockSpec((blk, D), lambda i: (i, 0)),
        out_shape=jax.ShapeDtypeStruct((N, D), jnp.float32),
    )(partials, partials, b2d)


def _lane_splat(vec, e):
    idx = jnp.full((16, 1), e, jnp.int32)
    return lax.gather(
        vec, idx,
        dimension_numbers=lax.GatherDimensionNumbers(
            offset_dims=(), collapsed_slice_dims=(0,), start_index_map=(0,)),
        slice_sizes=(1,),
        mode=lax.GatherScatterMode.PROMISE_IN_BOUNDS)


NBUF = 5
NGIF = 3


def _sc_body(h_hbm, src_hbm, dst_hbm, vals_hbm, zero_hbm, out_hbm,
             src_v, dst_v, vals_v, rows0, rows1, rows2, rows3, rows4,
             acc_sh, sem_g, sem_s):
    c = lax.axis_index("c")
    s = lax.axis_index("s")
    rows = [rows0, rows1, rows2, rows3, rows4]

    pltpu.sync_copy(zero_hbm.at[pl.ds(s * RPT, RPT)],
                    acc_sh.at[pl.ds(s * RPT, RPT)])
    pltpu.sync_copy(src_hbm.at[s], src_v)
    pltpu.sync_copy(dst_hbm.at[s], dst_v)
    pltpu.sync_copy(vals_hbm.at[s], vals_v)
    plsc.subcore_barrier()

    def start_gather(j, b):
        pltpu.async_copy(h_hbm.at[c].at[src_v.at[j]], rows[b], sem_g.at[b])

    def wait_gather(b):
        pltpu.make_async_copy(h_hbm.at[c].at[src_v.at[0]], rows[b],
                              sem_g.at[b]).wait()

    def start_scatter(j, b):
        pltpu.async_copy(rows[b], acc_sh.at[dst_v.at[j]], sem_s.at[b],
                         add=True)

    def wait_scatter(b):
        pltpu.make_async_copy(rows[b], acc_sh.at[dst_v.at[0]],
                              sem_s.at[b]).wait()

    def scale(j, b):
        @pl.loop(0, C // 16)
        def _grp(g):
            vgrp = vals_v[j, pl.ds(g * 16, 16)]
            for e in range(16):
                vsplat = _lane_splat(vgrp, e)
                row = g * 16 + e
                for d in range(DH // 16):
                    sl = pl.ds(d * 16, 16)
                    rows[b][row, sl] = rows[b][row, sl] * vsplat

    for b in range(NGIF):
        start_gather(b, b)

    @pl.loop(0, J // NBUF)
    def _iter(k):
        for b in range(NBUF):
            j = k * NBUF + b
            jn = j + NGIF
            bn = (b + NGIF) % NBUF

            @pl.when(jn < J)
            def _():
                @pl.when(jn >= NBUF)
                def _():
                    wait_scatter(bn)
                start_gather(jn, bn)

            wait_gather(b)
            scale(j, b)
            start_scatter(j, b)

    for b in range(NBUF):
        wait_scatter(b)

    plsc.subcore_barrier()
    pltpu.sync_copy(acc_sh.at[pl.ds(s * RPT, RPT)],
                    out_hbm.at[c, pl.ds(s * RPT, RPT)])


_sc_kernel = functools.partial(
    pl.kernel,
    out_type=jax.ShapeDtypeStruct((NC, N_PAD, DH), jnp.float32),
    mesh=plsc.VectorSubcoreMesh(core_axis_name="c", subcore_axis_name="s",
                                num_cores=NC, num_subcores=NS),
    compiler_params=pltpu.CompilerParams(use_tc_tiling_on_sc=False),
    scratch_types=[
        pltpu.VMEM((J, C), jnp.int32),
        pltpu.VMEM((J, C), jnp.int32),
        pltpu.VMEM((J, C), jnp.float32),
        pltpu.VMEM((C, DH), jnp.float32),
        pltpu.VMEM((C, DH), jnp.float32),
        pltpu.VMEM((C, DH), jnp.float32),
        pltpu.VMEM((C, DH), jnp.float32),
        pltpu.VMEM((C, DH), jnp.float32),
        pltpu.VMEM_SHARED((N_PAD, DH), jnp.float32),
        pltpu.SemaphoreType.DMA((NBUF,)),
        pltpu.SemaphoreType.DMA((NBUF,)),
    ],
)(_sc_body)


@jax.jit
def kernel(x, edge_index, edge_vals, W, b):
    h2 = _matmul_split(x, W)
    src = edge_index[0].reshape(NS, J, C)
    dst = edge_index[1].reshape(NS, J, C)
    vals = edge_vals.reshape(NS, J, C)
    zeros = jnp.zeros((N_PAD, DH), jnp.float32)
    partials = _sc_kernel(h2, src, dst, vals, zeros)
    return _combine(partials, b.reshape(1, D))


# speedup vs baseline: 11.1990x; 2.2309x over previous
import functools

import jax
import jax.numpy as jnp
from jax import lax
from jax.experimental import pallas as pl
from jax.experimental.pallas import tpu as pltpu
from jax.experimental.pallas import tpu_sc as plsc

N = 10000
E = 320000
D = 128
DH = D // 2

NC = 2
NS = 16
EPT = E // NS
C = 80
J = EPT // C
RPT = 632
N_PAD = RPT * NS


def _mm_body(x_ref, w_ref, o_ref):
    h = jnp.dot(x_ref[...], w_ref[...], preferred_element_type=jnp.float32)
    o_ref[0] = h[:, :DH]
    o_ref[1] = h[:, DH:]


def _matmul_split(x, W):
    blk = 1000
    grid = N // blk
    return pl.pallas_call(
        _mm_body,
        grid=(grid,),
        in_specs=[
            pl.BlockSpec((blk, D), lambda i: (i, 0)),
            pl.BlockSpec((D, D), lambda i: (0, 0)),
        ],
        out_specs=pl.BlockSpec((2, blk, DH), lambda i: (0, i, 0)),
        out_shape=jax.ShapeDtypeStruct((2, N, DH), jnp.float32),
    )(x, W)


def _combine_body(p0_ref, p1_ref, b_ref, o_ref):
    h = jnp.concatenate([p0_ref[0], p1_ref[0]], axis=1)
    o_ref[...] = jnp.maximum(h + b_ref[...], 0.0)


def _combine(partials, b2d):
    blk = 1000
    grid = N // blk
    return pl.pallas_call(
        _combine_body,
        grid=(grid,),
        in_specs=[
            pl.BlockSpec((1, blk, DH), lambda i: (0, i, 0)),
            pl.BlockSpec((1, blk, DH), lambda i: (1, i, 0)),
            pl.BlockSpec((1, D), lambda i: (0, 0)),
        ],
        out_specs=pl.BlockSpec((blk, D), lambda i: (i, 0)),
        out_shape=jax.ShapeDtypeStruct((N, D), jnp.float32),
    )(partials, partials, b2d)


def _lane_splat(vec, e):
    idx = jnp.full((16, 1), e, jnp.int32)
    return lax.gather(
        vec, idx,
        dimension_numbers=lax.GatherDimensionNumbers(
            offset_dims=(), collapsed_slice_dims=(0,), start_index_map=(0,)),
        slice_sizes=(1,),
        mode=lax.GatherScatterMode.PROMISE_IN_BOUNDS)


NBUF = 5
NGIF = 3


def _sc_body(h_hbm, src_hbm, dst_hbm, vals_hbm, zero_hbm, out_hbm,
             src_v, dst_v, vals_v, rows0, rows1, rows2, rows3, rows4,
             acc_sh, sem_g, sem_s):
    c = lax.axis_index("c")
    s = lax.axis_index("s")
    rows = [rows0, rows1, rows2, rows3, rows4]

    pltpu.sync_copy(zero_hbm.at[pl.ds(s * RPT, RPT)],
                    acc_sh.at[pl.ds(s * RPT, RPT)])
    pltpu.sync_copy(src_hbm.at[s], src_v)
    pltpu.sync_copy(dst_hbm.at[s], dst_v)
    pltpu.sync_copy(vals_hbm.at[s], vals_v)
    plsc.subcore_barrier()

    def start_gather(j, b):
        pltpu.async_copy(h_hbm.at[c].at[src_v.at[j]], rows[b], sem_g.at[b])

    def wait_gather(b):
        pltpu.make_async_copy(h_hbm.at[c].at[src_v.at[0]], rows[b],
                              sem_g.at[b]).wait()

    def start_scatter(j, b):
        pltpu.async_copy(rows[b], acc_sh.at[dst_v.at[j]], sem_s.at[b],
                         add=True)

    def wait_scatter(b):
        pltpu.make_async_copy(rows[b], acc_sh.at[dst_v.at[0]],
                              sem_s.at[b]).wait()

    def scale(j, b):
        @plsc.parallel_loop(0, C // 16)
        def _grp(g):
            vgrp = vals_v[j, pl.ds(g * 16, 16)]
            for e in range(16):
                vsplat = _lane_splat(vgrp, e)
                row = g * 16 + e
                for d in range(DH // 16):
                    sl = pl.ds(d * 16, 16)
                    rows[b][row, sl] = rows[b][row, sl] * vsplat

    for b in range(NGIF):
        start_gather(b, b)

    @pl.loop(0, J // NBUF)
    def _iter(k):
        for b in range(NBUF):
            j = k * NBUF + b
            jn = j + NGIF
            bn = (b + NGIF) % NBUF

            @pl.when(jn < J)
            def _():
                @pl.when(jn >= NBUF)
                def _():
                    wait_scatter(bn)
                start_gather(jn, bn)

            wait_gather(b)
            scale(j, b)
            start_scatter(j, b)

    for b in range(NBUF):
        wait_scatter(b)

    plsc.subcore_barrier()
    pltpu.sync_copy(acc_sh.at[pl.ds(s * RPT, RPT)],
                    out_hbm.at[c, pl.ds(s * RPT, RPT)])


_sc_kernel = functools.partial(
    pl.kernel,
    out_type=jax.ShapeDtypeStruct((NC, N_PAD, DH), jnp.float32),
    mesh=plsc.VectorSubcoreMesh(core_axis_name="c", subcore_axis_name="s",
                                num_cores=NC, num_subcores=NS),
    compiler_params=pltpu.CompilerParams(use_tc_tiling_on_sc=False),
    scratch_types=[
        pltpu.VMEM((J, C), jnp.int32),
        pltpu.VMEM((J, C), jnp.int32),
        pltpu.VMEM((J, C), jnp.float32),
        pltpu.VMEM((C, DH), jnp.float32),
        pltpu.VMEM((C, DH), jnp.float32),
        pltpu.VMEM((C, DH), jnp.float32),
        pltpu.VMEM((C, DH), jnp.float32),
        pltpu.VMEM((C, DH), jnp.float32),
        pltpu.VMEM_SHARED((N_PAD, DH), jnp.float32),
        pltpu.SemaphoreType.DMA((NBUF,)),
        pltpu.SemaphoreType.DMA((NBUF,)),
    ],
)(_sc_body)


@jax.jit
def kernel(x, edge_index, edge_vals, W, b):
    h2 = _matmul_split(x, W)
    src = edge_index[0].reshape(NS, J, C)
    dst = edge_index[1].reshape(NS, J, C)
    vals = edge_vals.reshape(NS, J, C)
    zeros = jnp.zeros((N_PAD, DH), jnp.float32)
    partials = _sc_kernel(h2, src, dst, vals, zeros)
    return _combine(partials, b.reshape(1, D))


# speedup vs baseline: 11.2882x; 1.0080x over previous
import functools

import jax
import jax.numpy as jnp
from jax import lax
from jax.experimental import pallas as pl
from jax.experimental.pallas import tpu as pltpu
from jax.experimental.pallas import tpu_sc as plsc

N = 10000
E = 320000
D = 128
DH = D // 2

NC = 2
NS = 16
EPT = E // NS
C = 80
J = EPT // C
RPT = 632
N_PAD = RPT * NS
RCH = 79

NBUF = 5
NGIF = 3


def _mm_body(x_ref, w_ref, o_ref):
    h = jnp.dot(x_ref[...], w_ref[...], preferred_element_type=jnp.float32)
    o_ref[0] = h[:, :DH]
    o_ref[1] = h[:, DH:]


def _matmul_split(x, W):
    blk = 2000
    grid = N // blk
    return pl.pallas_call(
        _mm_body,
        grid=(grid,),
        in_specs=[
            pl.BlockSpec((blk, D), lambda i: (i, 0)),
            pl.BlockSpec((D, D), lambda i: (0, 0)),
        ],
        out_specs=pl.BlockSpec((2, blk, DH), lambda i: (0, i, 0)),
        out_shape=jax.ShapeDtypeStruct((2, N, DH), jnp.float32),
    )(x, W)


def _lane_splat(vec, e):
    idx = jnp.full((16, 1), e, jnp.int32)
    return lax.gather(
        vec, idx,
        dimension_numbers=lax.GatherDimensionNumbers(
            offset_dims=(), collapsed_slice_dims=(0,), start_index_map=(0,)),
        slice_sizes=(1,),
        mode=lax.GatherScatterMode.PROMISE_IN_BOUNDS)


def _sc_body(h_hbm, src_hbm, dst_hbm, vals_hbm, b_hbm, out_hbm,
             src_v, dst_v, vals_v, rows0, rows1, rows2, rows3, rows4,
             b_v, acc_sh, sem_g, sem_s):
    c = lax.axis_index("c")
    s = lax.axis_index("s")
    rows = [rows0, rows1, rows2, rows3, rows4]

    @plsc.parallel_loop(0, RCH)
    def _z(r):
        for d in range(DH // 16):
            rows0[r, pl.ds(d * 16, 16)] = jnp.zeros((16,), jnp.float32)
    for i in range(RPT // RCH):
        pltpu.sync_copy(rows0.at[pl.ds(0, RCH)],
                        acc_sh.at[pl.ds(s * RPT + i * RCH, RCH)])
    pltpu.sync_copy(src_hbm.at[s], src_v)
    pltpu.sync_copy(dst_hbm.at[s], dst_v)
    pltpu.sync_copy(vals_hbm.at[s], vals_v)
    pltpu.sync_copy(b_hbm, b_v)
    plsc.subcore_barrier()

    def start_gather(j, b):
        pltpu.async_copy(h_hbm.at[c].at[src_v.at[j]], rows[b], sem_g.at[b])

    def wait_gather(b):
        pltpu.make_async_copy(h_hbm.at[c].at[src_v.at[0]], rows[b],
                              sem_g.at[b]).wait()

    def start_scatter(j, b):
        pltpu.async_copy(rows[b], acc_sh.at[dst_v.at[j]], sem_s.at[b],
                         add=True)

    def wait_scatter(b):
        pltpu.make_async_copy(rows[b], acc_sh.at[dst_v.at[0]],
                              sem_s.at[b]).wait()

    def scale(j, b):
        @plsc.parallel_loop(0, C // 16)
        def _grp(g):
            vgrp = vals_v[j, pl.ds(g * 16, 16)]
            for e in range(16):
                vsplat = _lane_splat(vgrp, e)
                row = g * 16 + e
                for d in range(DH // 16):
                    sl = pl.ds(d * 16, 16)
                    rows[b][row, sl] = rows[b][row, sl] * vsplat

    for b in range(NGIF):
        start_gather(b, b)

    @pl.loop(0, J // NBUF)
    def _iter(k):
        for b in range(NBUF):
            j = k * NBUF + b
            jn = j + NGIF
            bn = (b + NGIF) % NBUF

            @pl.when(jn < J)
            def _():
                @pl.when(jn >= NBUF)
                def _():
                    wait_scatter(bn)
                start_gather(jn, bn)

            wait_gather(b)
            scale(j, b)
            start_scatter(j, b)

    for b in range(NBUF):
        wait_scatter(b)

    plsc.subcore_barrier()

    bvec = [b_v[pl.ds(c * DH + d * 16, 16)] for d in range(DH // 16)]
    for i in range(RPT // RCH):
        row0 = s * RPT + i * RCH
        pltpu.sync_copy(acc_sh.at[pl.ds(row0, RCH)], rows0.at[pl.ds(0, RCH)])

        @plsc.parallel_loop(0, RCH)
        def _relu(r):
            for d in range(DH // 16):
                sl = pl.ds(d * 16, 16)
                rows0[r, sl] = jnp.maximum(rows0[r, sl] + bvec[d], 0.0)

        pltpu.sync_copy(rows0.at[pl.ds(0, RCH)],
                        out_hbm.at[c, pl.ds(row0, RCH)])


_sc_kernel = functools.partial(
    pl.kernel,
    out_type=jax.ShapeDtypeStruct((NC, N_PAD, DH), jnp.float32),
    mesh=plsc.VectorSubcoreMesh(core_axis_name="c", subcore_axis_name="s",
                                num_cores=NC, num_subcores=NS),
    compiler_params=pltpu.CompilerParams(use_tc_tiling_on_sc=False),
    scratch_types=[
        pltpu.VMEM((J, C), jnp.int32),
        pltpu.VMEM((J, C), jnp.int32),
        pltpu.VMEM((J, C), jnp.float32),
        pltpu.VMEM((C, DH), jnp.float32),
        pltpu.VMEM((C, DH), jnp.float32),
        pltpu.VMEM((C, DH), jnp.float32),
        pltpu.VMEM((C, DH), jnp.float32),
        pltpu.VMEM((C, DH), jnp.float32),
        pltpu.VMEM((D,), jnp.float32),
        pltpu.VMEM_SHARED((N_PAD, DH), jnp.float32),
        pltpu.SemaphoreType.DMA((NBUF,)),
        pltpu.SemaphoreType.DMA((NBUF,)),
    ],
)(_sc_body)


@jax.jit
def kernel(x, edge_index, edge_vals, W, b):
    h2 = _matmul_split(x, W)
    src = edge_index[0].reshape(NS, J, C)
    dst = edge_index[1].reshape(NS, J, C)
    vals = edge_vals.reshape(NS, J, C)
    halves = _sc_kernel(h2, src, dst, vals, b)
    return jnp.concatenate([halves[0, :N], halves[1, :N]], axis=1)


# speedup vs baseline: 12.6454x; 1.1202x over previous
import functools

import jax
import jax.numpy as jnp
from jax import lax
from jax.experimental import pallas as pl
from jax.experimental.pallas import tpu as pltpu
from jax.experimental.pallas import tpu_sc as plsc

N = 10000
E = 320000
D = 128
DH = D // 2

NC = 2
NS = 16
EPT = E // NS
C = 80
J = EPT // C
RPT = N // NS
RCHS = (80, 80, 80, 80, 80, 80, 80, 65)

NBUF = 5
NGIF = 3


def _mm_body(x_ref, w_ref, o_ref):
    h = jnp.dot(x_ref[...], w_ref[...], preferred_element_type=jnp.float32)
    o_ref[0] = h[:, :DH]
    o_ref[1] = h[:, DH:]


def _matmul_split(x, W):
    blk = 2000
    grid = N // blk
    return pl.pallas_call(
        _mm_body,
        grid=(grid,),
        in_specs=[
            pl.BlockSpec((blk, D), lambda i: (i, 0)),
            pl.BlockSpec((D, D), lambda i: (0, 0)),
        ],
        out_specs=pl.BlockSpec((2, blk, DH), lambda i: (0, i, 0)),
        out_shape=jax.ShapeDtypeStruct((2, N, DH), jnp.float32),
    )(x, W)


def _lane_splat(vec, e):
    idx = jnp.full((16, 1), e, jnp.int32)
    return lax.gather(
        vec, idx,
        dimension_numbers=lax.GatherDimensionNumbers(
            offset_dims=(), collapsed_slice_dims=(0,), start_index_map=(0,)),
        slice_sizes=(1,),
        mode=lax.GatherScatterMode.PROMISE_IN_BOUNDS)


def _sc_body(h_hbm, src_hbm, dst_hbm, vals_hbm, b_hbm, out_hbm,
             src_v, dst_v, vals_v, rows0, rows1, rows2, rows3, rows4,
             b_v, acc_sh, sem_g, sem_s):
    c = lax.axis_index("c")
    s = lax.axis_index("s")
    rows = [rows0, rows1, rows2, rows3, rows4]

    @plsc.parallel_loop(0, C)
    def _z(r):
        for d in range(DH // 16):
            rows0[r, pl.ds(d * 16, 16)] = jnp.zeros((16,), jnp.float32)
    row0 = s * RPT
    for rch in RCHS:
        pltpu.sync_copy(rows0.at[pl.ds(0, rch)],
                        acc_sh.at[pl.ds(row0, rch)])
        row0 += rch
    pltpu.sync_copy(src_hbm.at[s], src_v)
    pltpu.sync_copy(dst_hbm.at[s], dst_v)
    pltpu.sync_copy(vals_hbm.at[s], vals_v)
    pltpu.sync_copy(b_hbm, b_v)
    plsc.subcore_barrier()

    def start_gather(j, b):
        pltpu.async_copy(h_hbm.at[c].at[src_v.at[j]], rows[b], sem_g.at[b])

    def wait_gather(b):
        pltpu.make_async_copy(h_hbm.at[c].at[src_v.at[0]], rows[b],
                              sem_g.at[b]).wait()

    def start_scatter(j, b):
        pltpu.async_copy(rows[b], acc_sh.at[dst_v.at[j]], sem_s.at[b],
                         add=True)

    def wait_scatter(b):
        pltpu.make_async_copy(rows[b], acc_sh.at[dst_v.at[0]],
                              sem_s.at[b]).wait()

    def scale(j, b):
        @plsc.parallel_loop(0, C // 16)
        def _grp(g):
            vgrp = vals_v[j, pl.ds(g * 16, 16)]
            for e in range(16):
                vsplat = _lane_splat(vgrp, e)
                row = g * 16 + e
                for d in range(DH // 16):
                    sl = pl.ds(d * 16, 16)
                    rows[b][row, sl] = rows[b][row, sl] * vsplat

    for b in range(NGIF):
        start_gather(b, b)

    @pl.loop(0, J // NBUF)
    def _iter(k):
        for b in range(NBUF):
            j = k * NBUF + b
            jn = j + NGIF
            bn = (b + NGIF) % NBUF

            @pl.when(jn < J)
            def _():
                @pl.when(jn >= NBUF)
                def _():
                    wait_scatter(bn)
                start_gather(jn, bn)

            wait_gather(b)
            scale(j, b)
            start_scatter(j, b)

    for b in range(NBUF):
        wait_scatter(b)

    plsc.subcore_barrier()

    bvec = [b_v[pl.ds(c * DH + d * 16, 16)] for d in range(DH // 16)]
    row0 = s * RPT
    for rch in RCHS:
        pltpu.sync_copy(acc_sh.at[pl.ds(row0, rch)], rows0.at[pl.ds(0, rch)])

        @plsc.parallel_loop(0, rch)
        def _relu(r):
            for d in range(DH // 16):
                sl = pl.ds(d * 16, 16)
                rows0[r, sl] = jnp.maximum(rows0[r, sl] + bvec[d], 0.0)

        pltpu.sync_copy(rows0.at[pl.ds(0, rch)],
                        out_hbm.at[pl.ds(row0, rch), pl.ds(c * DH, DH)])
        row0 += rch


_sc_kernel = functools.partial(
    pl.kernel,
    out_type=jax.ShapeDtypeStruct((N, D), jnp.float32),
    mesh=plsc.VectorSubcoreMesh(core_axis_name="c", subcore_axis_name="s",
                                num_cores=NC, num_subcores=NS),
    compiler_params=pltpu.CompilerParams(use_tc_tiling_on_sc=False),
    scratch_types=[
        pltpu.VMEM((J, C), jnp.int32),
        pltpu.VMEM((J, C), jnp.int32),
        pltpu.VMEM((J, C), jnp.float32),
        pltpu.VMEM((C, DH), jnp.float32),
        pltpu.VMEM((C, DH), jnp.float32),
        pltpu.VMEM((C, DH), jnp.float32),
        pltpu.VMEM((C, DH), jnp.float32),
        pltpu.VMEM((C, DH), jnp.float32),
        pltpu.VMEM((D,), jnp.float32),
        pltpu.VMEM_SHARED((N, DH), jnp.float32),
        pltpu.SemaphoreType.DMA((NBUF,)),
        pltpu.SemaphoreType.DMA((NBUF,)),
    ],
)(_sc_body)


@jax.jit
def kernel(x, edge_index, edge_vals, W, b):
    h2 = _matmul_split(x, W)
    src = edge_index[0].reshape(NS, J, C)
    dst = edge_index[1].reshape(NS, J, C)
    vals = edge_vals.reshape(NS, J, C)
    return _sc_kernel(h2, src, dst, vals, b)
